# baseline (device time: 658257 ns/iter reference)
import jax
import jax.numpy as jnp
from jax import lax
from jax.experimental import pallas as pl
from jax.experimental.pallas import tpu as pltpu

N_DEV = 4
M, K, N = 4096, 1024, 8192
MC = M // N_DEV
NH = N // 2
OT = 1024
SUB = 4
MS = MC // SUB
CJ = 1024
CT = 512


def _body(x_ref, w_ref, scale_ref, out_ref, recv, xq, wq, cstage,
          ostage, send_sems, recv_sems, conv_sems, ostage_sems, credit_sems):
    d = lax.axis_index("i")
    scale = scale_ref[0, 0]
    DIRS = (1, -1)
    halfbase = (0, NH)
    nbr = [jnp.mod(d + 1, N_DEV), jnp.mod(d - 1, N_DEV)]
    peer_out = [nbr[0], nbr[1]]
    peer_in = [nbr[1], nbr[0]]

    def rows(c):
        return pl.ds(c * MC, MC)

    def convert(tiles):
        cps = {}

        def start(i):
            src, dst, rsl, csl, dt = tiles[i]
            cp = pltpu.make_async_copy(
                src.at[rsl, csl], cstage.at[i % 2], conv_sems.at[i % 2])
            cp.start()
            cps[i] = cp

        for i in range(min(2, len(tiles))):
            start(i)
        for i in range(len(tiles)):
            src, dst, rsl, csl, dt = tiles[i]
            cps[i].wait()
            dst[rsl, csl] = cstage[i % 2].astype(dt)
            if i + 2 < len(tiles):
                start(i + 2)

    def x_tiles(chunk):
        return [(x_ref, xq, pl.ds(chunk * MC + i * CT, CT), slice(None),
                 jnp.float8_e4m3fn) for i in range(MC // CT)]

    def w_tiles(k):
        return [(w_ref, wq, pl.ds(ir * CT, CT),
                 pl.ds(halfbase[k] + jc * 1024, 1024), jnp.float8_e5m2)
                for jc in range(NH // 1024) for ir in range(K // CT)]

    def chunk_dot(xrows, k, j):
        return lax.dot_general(
            xq[xrows, :], wq[:, pl.ds(halfbase[k] + j * CJ, CJ)],
            (((1,), (0,)), ((), ())),
            preferred_element_type=jnp.float32)

    def mk_rdma(k, t, b):
        S, D = t % 2, (t + 1) % 2
        return pltpu.make_async_remote_copy(
            src_ref=recv.at[k, S, pl.ds(b * MS, MS)],
            dst_ref=recv.at[k, D, pl.ds(b * MS, MS)],
            send_sem=send_sems.at[k, S, b],
            recv_sem=recv_sems.at[k, D, b],
            device_id=(peer_out[k],),
            device_id_type=pl.DeviceIdType.MESH)

    rdmas = [[[None] * SUB for _ in range(6)] for _ in range(2)]
    convert(x_tiles(d) + w_tiles(0) + w_tiles(1))

    for b in range(SUB):
        for k in range(2):
            for j in range(NH // CJ):
                recv[k, 0, pl.ds(b * MS, MS), j * CJ:(j + 1) * CJ] = (
                    chunk_dot(pl.ds(d * MC + b * MS, MS), k, j)
                    .astype(jnp.bfloat16))
            r = mk_rdma(k, 0, b)
            r.start()
            rdmas[k][0][b] = r
    convert(x_tiles(jnp.mod(d - 1, N_DEV))
            + x_tiles(jnp.mod(d + 1, N_DEV))
            + x_tiles(jnp.mod(d + 2, N_DEV)))

    last_ocp = [None, None]

    def epilogue_sub(k, slot, c, b):
        rs = pl.ds(b * MS, MS)
        for j in range(NH // OT):
            if last_ocp[k] is not None:
                last_ocp[k].wait()
            v = recv[k, slot, rs, j * OT:(j + 1) * OT].astype(jnp.float32)
            ostage[k, :, :] = jnp.maximum(v * scale, 0.0)
            cp = pltpu.make_async_copy(
                ostage.at[k],
                out_ref.at[pl.ds(c * MC + b * MS, MS),
                           pl.ds(halfbase[k] + j * OT, OT)],
                ostage_sems.at[k])
            cp.start()
            last_ocp[k] = cp

    for t in range(6):
        D = (t + 1) % 2
        for b in range(SUB):
            for k, dirn in enumerate(DIRS):
                rdmas[k][t][b].wait_recv()
                if t <= 2:
                    crt = jnp.mod(d - dirn * (t + 1), N_DEV)
                    xrows = pl.ds(crt * MC + b * MS, MS)
                    rs = pl.ds(b * MS, MS)
                    for j in range(NH // CJ):
                        cs = slice(j * CJ, (j + 1) * CJ)
                        acc = (recv[k, D, rs, cs].astype(jnp.float32)
                               + chunk_dot(xrows, k, j))
                        recv[k, D, rs, cs] = acc.astype(jnp.bfloat16)
                rdmas[k][t][b].wait_send()
                pl.semaphore_signal(credit_sems.at[k], inc=1,
                                    device_id=(peer_in[k],),
                                    device_id_type=pl.DeviceIdType.MESH)
                if t < 5:
                    pl.semaphore_wait(credit_sems.at[k], 1)
                    r = mk_rdma(k, t + 1, b)
                    r.start()
                    rdmas[k][t + 1][b] = r
                if t >= 2:
                    c = (jnp.mod(d + dirn, N_DEV) if t == 2
                         else jnp.mod(d - dirn * (t - 3), N_DEV))
                    epilogue_sub(k, D, c, b)
    for k in range(2):
        if last_ocp[k] is not None:
            last_ocp[k].wait()
        pl.semaphore_wait(credit_sems.at[k], SUB)


def kernel(x, w_mat, scale_x, scale_w):
    scale = (scale_x * scale_w).reshape(1, 1).astype(jnp.float32)
    return pl.pallas_call(
        _body,
        in_specs=[
            pl.BlockSpec(memory_space=pl.ANY),
            pl.BlockSpec(memory_space=pl.ANY),
            pl.BlockSpec(memory_space=pltpu.SMEM),
        ],
        out_specs=pl.BlockSpec(memory_space=pl.ANY),
        out_shape=jax.ShapeDtypeStruct((M, N), jnp.float32),
        scratch_shapes=[
            pltpu.VMEM((2, 2, MC, NH), jnp.bfloat16),
            pltpu.VMEM((M, K), jnp.float8_e4m3fn),
            pltpu.VMEM((K, N), jnp.float8_e5m2),
            pltpu.VMEM((2, CT, 1024), jnp.float32),
            pltpu.VMEM((2, MS, OT), jnp.float32),
            pltpu.SemaphoreType.DMA((2, 2, SUB)),
            pltpu.SemaphoreType.DMA((2, 2, SUB)),
            pltpu.SemaphoreType.DMA((2,)),
            pltpu.SemaphoreType.DMA((2,)),
            pltpu.SemaphoreType.REGULAR((2,)),
        ],
        compiler_params=pltpu.CompilerParams(
            vmem_limit_bytes=64 * 1024 * 1024),
    )(x, w_mat, scale)


# device time: 655706 ns/iter; 1.0039x vs baseline; 1.0039x over previous
import jax
import jax.numpy as jnp
from jax import lax
from jax.experimental import pallas as pl
from jax.experimental.pallas import tpu as pltpu

N_DEV = 4
M, K, N = 4096, 1024, 8192
MC = M // N_DEV
NH = N // 2
OT = 1024
SUB = 4
MS = MC // SUB
CJ = 1024
CT = 512


def _body(x_ref, w_ref, scale_ref, out_ref, recv, xq, wq, cstage,
          ostage, send_sems, recv_sems, conv_sems, ostage_sems, credit_sems):
    d = lax.axis_index("i")
    scale = scale_ref[0, 0]
    DIRS = (1, -1)
    halfbase = (0, NH)
    nbr = [jnp.mod(d + 1, N_DEV), jnp.mod(d - 1, N_DEV)]
    peer_out = [nbr[0], nbr[1]]
    peer_in = [nbr[1], nbr[0]]

    def rows(c):
        return pl.ds(c * MC, MC)

    def convert(tiles):
        cps = {}

        def start(i):
            src, dst, rsl, csl, dt = tiles[i]
            cp = pltpu.make_async_copy(
                src.at[rsl, csl], cstage.at[i % 2], conv_sems.at[i % 2])
            cp.start()
            cps[i] = cp

        for i in range(min(2, len(tiles))):
            start(i)
        for i in range(len(tiles)):
            src, dst, rsl, csl, dt = tiles[i]
            cps[i].wait()
            dst[rsl, csl] = cstage[i % 2].astype(dt)
            if i + 2 < len(tiles):
                start(i + 2)

    def x_tiles(chunk):
        return [(x_ref, xq, pl.ds(chunk * MC + i * CT, CT), slice(None),
                 jnp.float8_e4m3fn) for i in range(MC // CT)]

    def w_tiles(k):
        return [(w_ref, wq, pl.ds(ir * CT, CT),
                 pl.ds(halfbase[k] + jc * 1024, 1024), jnp.float8_e5m2)
                for jc in range(NH // 1024) for ir in range(K // CT)]

    def chunk_dot(xrows, k, j):
        return lax.dot_general(
            xq[xrows, :], wq[:, pl.ds(halfbase[k] + j * CJ, CJ)],
            (((1,), (0,)), ((), ())),
            preferred_element_type=jnp.float32)

    def mk_rdma(k, t, b):
        S, D = t % 2, (t + 1) % 2
        return pltpu.make_async_remote_copy(
            src_ref=recv.at[k, S, pl.ds(b * MS, MS)],
            dst_ref=recv.at[k, D, pl.ds(b * MS, MS)],
            send_sem=send_sems.at[k, S, b],
            recv_sem=recv_sems.at[k, D, b],
            device_id=(peer_out[k],),
            device_id_type=pl.DeviceIdType.MESH)

    rdmas = [[[None] * SUB for _ in range(6)] for _ in range(2)]
    convert(x_tiles(d) + w_tiles(0) + w_tiles(1))

    barrier = pltpu.get_barrier_semaphore()
    for k in range(2):
        pl.semaphore_signal(barrier, inc=1, device_id=(nbr[k],),
                            device_id_type=pl.DeviceIdType.MESH)
    pl.semaphore_wait(barrier, 2)

    for b in range(SUB):
        for k in range(2):
            for j in range(NH // CJ):
                recv[k, 0, pl.ds(b * MS, MS), j * CJ:(j + 1) * CJ] = (
                    chunk_dot(pl.ds(d * MC + b * MS, MS), k, j)
                    .astype(jnp.bfloat16))
            r = mk_rdma(k, 0, b)
            r.start()
            rdmas[k][0][b] = r
    convert(x_tiles(jnp.mod(d - 1, N_DEV))
            + x_tiles(jnp.mod(d + 1, N_DEV))
            + x_tiles(jnp.mod(d + 2, N_DEV)))

    last_ocp = [None, None]

    def epilogue_sub(k, slot, c, b):
        rs = pl.ds(b * MS, MS)
        for j in range(NH // OT):
            if last_ocp[k] is not None:
                last_ocp[k].wait()
            v = recv[k, slot, rs, j * OT:(j + 1) * OT].astype(jnp.float32)
            ostage[k, :, :] = jnp.maximum(v * scale, 0.0)
            cp = pltpu.make_async_copy(
                ostage.at[k],
                out_ref.at[pl.ds(c * MC + b * MS, MS),
                           pl.ds(halfbase[k] + j * OT, OT)],
                ostage_sems.at[k])
            cp.start()
            last_ocp[k] = cp

    for t in range(6):
        D = (t + 1) % 2
        for b in range(SUB):
            for k, dirn in enumerate(DIRS):
                rdmas[k][t][b].wait_recv()
                if t <= 2:
                    crt = jnp.mod(d - dirn * (t + 1), N_DEV)
                    xrows = pl.ds(crt * MC + b * MS, MS)
                    rs = pl.ds(b * MS, MS)
                    for j in range(NH // CJ):
                        cs = slice(j * CJ, (j + 1) * CJ)
                        acc = (recv[k, D, rs, cs].astype(jnp.float32)
                               + chunk_dot(xrows, k, j))
                        recv[k, D, rs, cs] = acc.astype(jnp.bfloat16)
                rdmas[k][t][b].wait_send()
                pl.semaphore_signal(credit_sems.at[k], inc=1,
                                    device_id=(peer_in[k],),
                                    device_id_type=pl.DeviceIdType.MESH)
                if t < 5:
                    pl.semaphore_wait(credit_sems.at[k], 1)
                    r = mk_rdma(k, t + 1, b)
                    r.start()
                    rdmas[k][t + 1][b] = r
                if t >= 2:
                    c = (jnp.mod(d + dirn, N_DEV) if t == 2
                         else jnp.mod(d - dirn * (t - 3), N_DEV))
                    epilogue_sub(k, D, c, b)
    for k in range(2):
        if last_ocp[k] is not None:
            last_ocp[k].wait()
        pl.semaphore_wait(credit_sems.at[k], SUB)


def kernel(x, w_mat, scale_x, scale_w):
    scale = (scale_x * scale_w).reshape(1, 1).astype(jnp.float32)
    return pl.pallas_call(
        _body,
        in_specs=[
            pl.BlockSpec(memory_space=pl.ANY),
            pl.BlockSpec(memory_space=pl.ANY),
            pl.BlockSpec(memory_space=pltpu.SMEM),
        ],
        out_specs=pl.BlockSpec(memory_space=pl.ANY),
        out_shape=jax.ShapeDtypeStruct((M, N), jnp.float32),
        scratch_shapes=[
            pltpu.VMEM((2, 2, MC, NH), jnp.bfloat16),
            pltpu.VMEM((M, K), jnp.float8_e4m3fn),
            pltpu.VMEM((K, N), jnp.float8_e5m2),
            pltpu.VMEM((2, CT, 1024), jnp.float32),
            pltpu.VMEM((2, MS, OT), jnp.float32),
            pltpu.SemaphoreType.DMA((2, 2, SUB)),
            pltpu.SemaphoreType.DMA((2, 2, SUB)),
            pltpu.SemaphoreType.DMA((2,)),
            pltpu.SemaphoreType.DMA((2,)),
            pltpu.SemaphoreType.REGULAR((2,)),
        ],
        compiler_params=pltpu.CompilerParams(
            collective_id=0, vmem_limit_bytes=64 * 1024 * 1024),
    )(x, w_mat, scale)
